# initial kernel scaffold (unmeasured)
import jax
import jax.numpy as jnp
from jax import lax
from jax.experimental import pallas as pl
from jax.experimental.pallas import tpu as pltpu

N_DEV = 8


def kernel(x, w_mat):
    m_per, k = x.shape
    n = w_mat.shape[1]
    n_per = n // N_DEV

    def body(x_ref, w_ref, out_ref, xb_ref, y_ref, send_sems, recv_sems):
        my = lax.axis_index("i")

        barrier = pltpu.get_barrier_semaphore()
        for d in range(1, N_DEV):
            pl.semaphore_signal(
                barrier, inc=1,
                device_id=((my + d) % N_DEV,),
                device_id_type=pl.DeviceIdType.MESH,
            )
        pl.semaphore_wait(barrier, N_DEV - 1)

        xb_ref[:, :] = x_ref[:, :].astype(jnp.bfloat16)

        for j in range(N_DEV):
            wb = w_ref[:, j * n_per:(j + 1) * n_per].astype(jnp.bfloat16)
            y_ref[j, :, :] = jnp.dot(
                xb_ref[:, :], wb, preferred_element_type=jnp.float32
            )

            @pl.when(j == my)
            def _():
                out_ref[pl.ds(my * m_per, m_per), :] = y_ref[j, :, :]

            @pl.when(j != my)
            def _():
                rdma = pltpu.make_async_remote_copy(
                    src_ref=y_ref.at[j],
                    dst_ref=out_ref.at[pl.ds(my * m_per, m_per), :],
                    send_sem=send_sems.at[j],
                    recv_sem=recv_sems.at[my],
                    device_id=(j,),
                    device_id_type=pl.DeviceIdType.MESH,
                )
                rdma.start()

        for s in range(N_DEV):
            @pl.when(s != my)
            def _():
                desc = pltpu.make_async_remote_copy(
                    src_ref=y_ref.at[s],
                    dst_ref=out_ref.at[pl.ds(s * m_per, m_per), :],
                    send_sem=send_sems.at[s],
                    recv_sem=recv_sems.at[s],
                    device_id=(s,),
                    device_id_type=pl.DeviceIdType.MESH,
                )
                desc.wait_recv()
                desc.wait_send()

    return pl.pallas_call(
        body,
        out_shape=jax.ShapeDtypeStruct((N_DEV * m_per, n_per), jnp.float32),
        in_specs=[
            pl.BlockSpec(memory_space=pltpu.VMEM),
            pl.BlockSpec(memory_space=pltpu.VMEM),
        ],
        out_specs=pl.BlockSpec(memory_space=pltpu.VMEM),
        scratch_shapes=[
            pltpu.VMEM((m_per, k), jnp.bfloat16),
            pltpu.VMEM((N_DEV, m_per, n_per), jnp.float32),
            pltpu.SemaphoreType.DMA((N_DEV,)),
            pltpu.SemaphoreType.DMA((N_DEV,)),
        ],
        compiler_params=pltpu.CompilerParams(collective_id=0),
    )(x, w_mat)


# baseline (device time: 64000 ns/iter reference)
import jax
import jax.numpy as jnp
from jax import lax
from jax.experimental import pallas as pl
from jax.experimental.pallas import tpu as pltpu

N_DEV = 8


def kernel(x, w_mat):
    m_per, k = x.shape
    n = w_mat.shape[1]
    n_per = n // N_DEV

    def body(x_ref, w_ref, out_ref, xb_ref, y_ref, send_sems, recv_sems):
        my = lax.axis_index("i")

        barrier = pltpu.get_barrier_semaphore()
        for d in range(1, N_DEV):
            pl.semaphore_signal(
                barrier, inc=1,
                device_id=((my + d) % N_DEV,),
                device_id_type=pl.DeviceIdType.MESH,
            )
        pl.semaphore_wait(barrier, N_DEV - 1)

        xb_ref[:, :] = x_ref[:, :].astype(jnp.bfloat16)

        for j in range(N_DEV):
            wb = w_ref[:, j * n_per:(j + 1) * n_per].astype(jnp.bfloat16)
            y_ref[j, :, :] = jnp.dot(
                xb_ref[:, :], wb, preferred_element_type=jnp.float32
            )

            @pl.when(j == my)
            def _():
                out_ref[pl.ds(my * m_per, m_per), :] = y_ref[j, :, :]

            @pl.when(j != my)
            def _():
                rdma = pltpu.make_async_remote_copy(
                    src_ref=y_ref.at[j],
                    dst_ref=out_ref.at[pl.ds(my * m_per, m_per), :],
                    send_sem=send_sems.at[j],
                    recv_sem=recv_sems.at[my],
                    device_id=(j,),
                    device_id_type=pl.DeviceIdType.MESH,
                )
                rdma.start()

        for s in range(N_DEV):
            @pl.when(s != my)
            def _():
                desc = pltpu.make_async_remote_copy(
                    src_ref=y_ref.at[s],
                    dst_ref=out_ref.at[pl.ds(s * m_per, m_per), :],
                    send_sem=send_sems.at[s],
                    recv_sem=recv_sems.at[s],
                    device_id=(s,),
                    device_id_type=pl.DeviceIdType.MESH,
                )
                desc.wait_recv()
                desc.wait_send()

    return pl.pallas_call(
        body,
        out_shape=jax.ShapeDtypeStruct((N_DEV * m_per, n_per), jnp.float32),
        in_specs=[
            pl.BlockSpec(memory_space=pltpu.VMEM),
            pl.BlockSpec(memory_space=pltpu.VMEM),
        ],
        out_specs=pl.BlockSpec(memory_space=pltpu.VMEM),
        scratch_shapes=[
            pltpu.VMEM((m_per, k), jnp.bfloat16),
            pltpu.VMEM((N_DEV, m_per, n_per), jnp.float32),
            pltpu.SemaphoreType.DMA((N_DEV,)),
            pltpu.SemaphoreType.DMA((N_DEV,)),
        ],
        compiler_params=pltpu.CompilerParams(
            collective_id=0,
            vmem_limit_bytes=100 * 1024 * 1024,
        ),
    )(x, w_mat)


# device time: 56709 ns/iter; 1.1286x vs baseline; 1.1286x over previous
import jax
import jax.numpy as jnp
from jax import lax
from jax.experimental import pallas as pl
from jax.experimental.pallas import tpu as pltpu

N_DEV = 8


def kernel(x, w_mat):
    m_per, k = x.shape
    n = w_mat.shape[1]
    n_per = n // N_DEV

    def body(x_ref, w_ref, out_ref, xb_ref, y_ref, rb_ref, send_sems, recv_sems):
        my = lax.axis_index("i")

        barrier = pltpu.get_barrier_semaphore()
        for d in range(1, N_DEV):
            pl.semaphore_signal(
                barrier, inc=1,
                device_id=((my + d) % N_DEV,),
                device_id_type=pl.DeviceIdType.MESH,
            )

        xb_ref[:, :] = x_ref[:, :].astype(jnp.bfloat16)

        for j in range(N_DEV):
            wb = w_ref[:, j * n_per:(j + 1) * n_per].astype(jnp.bfloat16)
            blk = jnp.dot(xb_ref[:, :], wb, preferred_element_type=jnp.float32)

            @pl.when(j == my)
            def _():
                out_ref[pl.ds(my * m_per, m_per), :] = blk

            @pl.when(j != my)
            def _():
                y_ref[j, :, :] = blk.astype(jnp.bfloat16)

        pl.semaphore_wait(barrier, N_DEV - 1)
        for d in range(1, N_DEV):
            t = (my + d) % N_DEV
            rdma = pltpu.make_async_remote_copy(
                src_ref=y_ref.at[t],
                dst_ref=rb_ref.at[my],
                send_sem=send_sems.at[d],
                recv_sem=recv_sems.at[my],
                device_id=(t,),
                device_id_type=pl.DeviceIdType.MESH,
            )
            rdma.start()

        for d in range(1, N_DEV):
            s = (my - d) % N_DEV
            desc = pltpu.make_async_remote_copy(
                src_ref=y_ref.at[s],
                dst_ref=rb_ref.at[s],
                send_sem=send_sems.at[d],
                recv_sem=recv_sems.at[s],
                device_id=(s,),
                device_id_type=pl.DeviceIdType.MESH,
            )
            desc.wait_recv()
            out_ref[pl.ds(s * m_per, m_per), :] = rb_ref[s].astype(jnp.float32)

        for d in range(1, N_DEV):
            desc = pltpu.make_async_remote_copy(
                src_ref=y_ref.at[(my + d) % N_DEV],
                dst_ref=rb_ref.at[my],
                send_sem=send_sems.at[d],
                recv_sem=recv_sems.at[my],
                device_id=((my + d) % N_DEV,),
                device_id_type=pl.DeviceIdType.MESH,
            )
            desc.wait_send()

    return pl.pallas_call(
        body,
        out_shape=jax.ShapeDtypeStruct((N_DEV * m_per, n_per), jnp.float32),
        in_specs=[
            pl.BlockSpec(memory_space=pltpu.VMEM),
            pl.BlockSpec(memory_space=pltpu.VMEM),
        ],
        out_specs=pl.BlockSpec(memory_space=pltpu.VMEM),
        scratch_shapes=[
            pltpu.VMEM((m_per, k), jnp.bfloat16),
            pltpu.VMEM((N_DEV, m_per, n_per), jnp.bfloat16),
            pltpu.VMEM((N_DEV, m_per, n_per), jnp.bfloat16),
            pltpu.SemaphoreType.DMA((N_DEV,)),
            pltpu.SemaphoreType.DMA((N_DEV,)),
        ],
        compiler_params=pltpu.CompilerParams(
            collective_id=0,
            vmem_limit_bytes=100 * 1024 * 1024,
        ),
    )(x, w_mat)


# device time: 39052 ns/iter; 1.6388x vs baseline; 1.4521x over previous
import jax
import jax.numpy as jnp
from jax import lax
from jax.experimental import pallas as pl
from jax.experimental.pallas import tpu as pltpu

N_DEV = 8


def kernel(x, w_mat):
    m_per, k = x.shape
    n = w_mat.shape[1]
    n_per = n // N_DEV

    def body(x_ref, w_hbm, out_ref, xb_ref, wslab_ref, y_ref, rb_ref,
             wdma_sems, send_sems, recv_sems):
        my = lax.axis_index("i")

        barrier = pltpu.get_barrier_semaphore()
        for d in range(1, N_DEV):
            pl.semaphore_signal(
                barrier, inc=1,
                device_id=((my + d) % N_DEV,),
                device_id_type=pl.DeviceIdType.MESH,
            )

        def wslab_copy(d):
            t = (my + d) % N_DEV
            return pltpu.make_async_copy(
                w_hbm.at[:, pl.ds(t * n_per, n_per)],
                wslab_ref.at[d % 2],
                wdma_sems.at[d % 2],
            )

        wslab_copy(0).start()
        wslab_copy(1).start()

        xb_ref[:, :] = x_ref[:, :].astype(jnp.bfloat16)

        for d in range(N_DEV):
            t = (my + d) % N_DEV
            wslab_copy(d).wait()
            wb = wslab_ref[d % 2].astype(jnp.bfloat16)
            blk = jnp.dot(xb_ref[:, :], wb, preferred_element_type=jnp.float32)
            if d + 2 < N_DEV:
                wslab_copy(d + 2).start()

            if d == 0:
                out_ref[pl.ds(my * m_per, m_per), :] = blk
                pl.semaphore_wait(barrier, N_DEV - 1)
            else:
                y_ref[d, :, :] = blk.astype(jnp.bfloat16)
                rdma = pltpu.make_async_remote_copy(
                    src_ref=y_ref.at[d],
                    dst_ref=rb_ref.at[my],
                    send_sem=send_sems.at[d],
                    recv_sem=recv_sems.at[my],
                    device_id=(t,),
                    device_id_type=pl.DeviceIdType.MESH,
                )
                rdma.start()

        for d in range(1, N_DEV):
            s = (my - d) % N_DEV
            desc = pltpu.make_async_remote_copy(
                src_ref=y_ref.at[d],
                dst_ref=rb_ref.at[s],
                send_sem=send_sems.at[d],
                recv_sem=recv_sems.at[s],
                device_id=(s,),
                device_id_type=pl.DeviceIdType.MESH,
            )
            desc.wait_recv()
            out_ref[pl.ds(s * m_per, m_per), :] = rb_ref[s].astype(jnp.float32)

        for d in range(1, N_DEV):
            desc = pltpu.make_async_remote_copy(
                src_ref=y_ref.at[d],
                dst_ref=rb_ref.at[my],
                send_sem=send_sems.at[d],
                recv_sem=recv_sems.at[my],
                device_id=((my + d) % N_DEV,),
                device_id_type=pl.DeviceIdType.MESH,
            )
            desc.wait_send()

    return pl.pallas_call(
        body,
        out_shape=jax.ShapeDtypeStruct((N_DEV * m_per, n_per), jnp.float32),
        in_specs=[
            pl.BlockSpec(memory_space=pltpu.VMEM),
            pl.BlockSpec(memory_space=pltpu.MemorySpace.HBM),
        ],
        out_specs=pl.BlockSpec(memory_space=pltpu.VMEM),
        scratch_shapes=[
            pltpu.VMEM((m_per, k), jnp.bfloat16),
            pltpu.VMEM((2, k, n_per), jnp.float32),
            pltpu.VMEM((N_DEV, m_per, n_per), jnp.bfloat16),
            pltpu.VMEM((N_DEV, m_per, n_per), jnp.bfloat16),
            pltpu.SemaphoreType.DMA((2,)),
            pltpu.SemaphoreType.DMA((N_DEV,)),
            pltpu.SemaphoreType.DMA((N_DEV,)),
        ],
        compiler_params=pltpu.CompilerParams(
            collective_id=0,
            vmem_limit_bytes=100 * 1024 * 1024,
        ),
    )(x, w_mat)
